# Initial kernel scaffold; baseline (speedup 1.0000x reference)
import functools

import jax
import jax.numpy as jnp
from jax import lax
from jax.experimental import pallas as pl
from jax.experimental.pallas import tpu as pltpu
from jax.experimental.pallas import tpu_sc as plsc

N = 10000
E = 320000
D = 128
NC = 2
NS = 16
NW = NC * NS
EPT = E // NW
B = 80
NB = EPT // B
ROWS_PER_TILE = N // NS
RCHUNK = 125
CW = 16


def _sc_segment_sum(x, src, dst):
    mesh = plsc.VectorSubcoreMesh(core_axis_name="c", subcore_axis_name="s")

    @functools.partial(
        pl.kernel,
        out_type=[
            jax.ShapeDtypeStruct((NC, N, D), jnp.float32),
            jax.ShapeDtypeStruct((NC, N, CW), jnp.float32),
        ],
        mesh=mesh,
        scratch_types=[
            pltpu.VMEM((B,), jnp.int32),
            pltpu.VMEM((B,), jnp.int32),
            pltpu.VMEM((B, D), jnp.float32),
            pltpu.VMEM((B, CW), jnp.float32),
            pltpu.VMEM((RCHUNK, D), jnp.float32),
            pltpu.VMEM((RCHUNK, CW), jnp.float32),
            pltpu.VMEM_SHARED((N, D), jnp.float32),
            pltpu.VMEM_SHARED((N, CW), jnp.float32),
            pltpu.SemaphoreType.DMA,
        ],
    )
    def k(x_hbm, src_hbm, dst_hbm, sum_hbm, cnt_hbm,
          sidx, didx, rows, ones, zrows, zcnt, acc, cnt, sem):
        c = lax.axis_index("c")
        s = lax.axis_index("s")
        wid = c * NS + s

        @pl.loop(0, B)
        def _(i):
            ones[i, :] = jnp.ones((CW,), jnp.float32)

        @pl.loop(0, RCHUNK)
        def _(i):
            zcnt[i, :] = jnp.zeros((CW,), jnp.float32)

            @pl.loop(0, D // 16)
            def _(j):
                zrows[i, pl.ds(j * 16, 16)] = jnp.zeros((16,), jnp.float32)

        row0 = s * ROWS_PER_TILE
        for t in range(ROWS_PER_TILE // RCHUNK):
            pltpu.sync_copy(zrows, acc.at[pl.ds(row0 + t * RCHUNK, RCHUNK)])
            pltpu.sync_copy(zcnt, cnt.at[pl.ds(row0 + t * RCHUNK, RCHUNK)])
        plsc.subcore_barrier()

        base = wid * EPT

        @pl.loop(0, NB)
        def _(j):
            off = base + j * B
            pltpu.sync_copy(src_hbm.at[pl.ds(off, B)], sidx)
            pltpu.sync_copy(dst_hbm.at[pl.ds(off, B)], didx)
            pltpu.async_copy(x_hbm.at[sidx], rows, sem).wait()
            pltpu.sync_copy(rows, acc.at[didx], add=True)
            pltpu.sync_copy(ones, cnt.at[didx], add=True)

        plsc.subcore_barrier()

        for t in range(ROWS_PER_TILE // RCHUNK):
            r = row0 + t * RCHUNK
            pltpu.sync_copy(acc.at[pl.ds(r, RCHUNK)], zrows)
            pltpu.sync_copy(zrows, sum_hbm.at[c, pl.ds(r, RCHUNK)])
            pltpu.sync_copy(cnt.at[pl.ds(r, RCHUNK)], zcnt)
            pltpu.sync_copy(zcnt, cnt_hbm.at[c, pl.ds(r, RCHUNK)])

    return k(x, src, dst)


def _finish(x, p0, p1, c0, c1, W_l, b_l, W_r):
    R = 1000
    dn = (((1,), (1,)), ((), ()))

    def body(p0_ref, p1_ref, c0_ref, c1_ref, x_ref, wl_ref, wr_ref, b_ref,
             o_ref):
        cnt = jnp.maximum(c0_ref[:, 0:1] + c1_ref[:, 0:1], 1.0)
        agg = (p0_ref[...] + p1_ref[...]) / cnt
        acc = lax.dot_general(agg, wl_ref[...], dn,
                              precision=lax.Precision.HIGHEST,
                              preferred_element_type=jnp.float32)
        acc = acc + lax.dot_general(x_ref[...], wr_ref[...], dn,
                                    precision=lax.Precision.HIGHEST,
                                    preferred_element_type=jnp.float32)
        o_ref[...] = jnp.maximum(acc + b_ref[...], 0.0)

    return pl.pallas_call(
        body,
        grid=(N // R,),
        in_specs=[
            pl.BlockSpec((R, D), lambda i: (i, 0)),
            pl.BlockSpec((R, D), lambda i: (i, 0)),
            pl.BlockSpec((R, CW), lambda i: (i, 0)),
            pl.BlockSpec((R, CW), lambda i: (i, 0)),
            pl.BlockSpec((R, D), lambda i: (i, 0)),
            pl.BlockSpec((D, D), lambda i: (0, 0)),
            pl.BlockSpec((D, D), lambda i: (0, 0)),
            pl.BlockSpec((1, D), lambda i: (0, 0)),
        ],
        out_specs=pl.BlockSpec((R, D), lambda i: (i, 0)),
        out_shape=jax.ShapeDtypeStruct((N, D), jnp.float32),
    )(p0, p1, c0, c1, x, W_l, W_r, b_l.reshape(1, D))


def kernel(x, adj, W_l, b_l, W_r):
    src = adj[0].astype(jnp.int32)
    dst = adj[1].astype(jnp.int32)
    sums, cnts = _sc_segment_sum(x, src, dst)
    return _finish(x, sums[0], sums[1], cnts[0], cnts[1], W_l, b_l, W_r)


# speedup vs baseline: 5.2598x; 5.2598x over previous
import dataclasses
import functools

import jax
import jax.numpy as jnp
from jax import lax
from jax.experimental import pallas as pl
from jax.experimental.pallas import tpu as pltpu
from jax.experimental.pallas import tpu_sc as plsc

N = 10000
NPAD = 10240
E = 320000
D = 128
NC = 2
NS = 16
NW = NC * NS
EPT = E // NW
B = 80
NB = EPT // B
ROWS_PER_TILE = NPAD // NS
RCHUNK = 128
CROWS = NPAD // 8
CPT = CROWS // NS


def _sc_compiler_params():
    cp = pltpu.CompilerParams()
    if "needs_layout_passes" in pltpu.CompilerParams.__dataclass_fields__:
        cp = dataclasses.replace(cp, needs_layout_passes=False)
    return cp


def _sc_segment_sum(x, src, dst):
    mesh = plsc.VectorSubcoreMesh(core_axis_name="c", subcore_axis_name="s")

    @functools.partial(
        pl.kernel,
        compiler_params=_sc_compiler_params(),
        out_type=[
            jax.ShapeDtypeStruct((NC, NPAD, D), jnp.float32),
            jax.ShapeDtypeStruct((NC, CROWS, D), jnp.float32),
        ],
        mesh=mesh,
        scratch_types=[
            pltpu.VMEM((B,), jnp.int32),
            pltpu.VMEM((B,), jnp.int32),
            pltpu.VMEM((B,), jnp.int32),
            pltpu.VMEM((B, D), jnp.float32),
            pltpu.VMEM((B, D), jnp.float32),
            pltpu.VMEM((RCHUNK, D), jnp.float32),
            pltpu.VMEM_SHARED((NPAD, D), jnp.float32),
            pltpu.VMEM_SHARED((CROWS, D), jnp.float32),
            pltpu.SemaphoreType.DMA,
        ],
    )
    def k(x_hbm, src_hbm, dst_hbm, sum_hbm, cnt_hbm,
          sidx, didx, cidx, rows, pay, zrows, acc, cacc, sem):
        c = lax.axis_index("c")
        s = lax.axis_index("s")
        wid = c * NS + s
        ones16 = jnp.ones((16,), jnp.float32)
        zeros16 = jnp.zeros((16,), jnp.float32)
        iota16 = lax.iota(jnp.int32, 16)

        @pl.loop(0, B)
        def _(i):
            @pl.loop(0, D // 16)
            def _(j):
                pay[i, pl.ds(j * 16, 16)] = zeros16

        @pl.loop(0, RCHUNK)
        def _(i):
            @pl.loop(0, D // 16)
            def _(j):
                zrows[i, pl.ds(j * 16, 16)] = zeros16

        row0 = s * ROWS_PER_TILE
        for t in range(ROWS_PER_TILE // RCHUNK):
            pltpu.sync_copy(zrows, acc.at[pl.ds(row0 + t * RCHUNK, RCHUNK)])
        pltpu.sync_copy(zrows.at[pl.ds(0, CPT)],
                        cacc.at[pl.ds(s * CPT, CPT)])
        plsc.subcore_barrier()

        base = wid * EPT

        @pl.loop(0, NB)
        def _(j):
            off = base + j * B
            pltpu.sync_copy(src_hbm.at[pl.ds(off, B)], sidx)
            pltpu.sync_copy(dst_hbm.at[pl.ds(off, B)], didx)
            pltpu.async_copy(x_hbm.at[sidx], rows, sem).wait()

            @pl.loop(0, B // 16)
            def _(kk):
                dv = didx[pl.ds(kk * 16, 16)]
                cidx[pl.ds(kk * 16, 16)] = lax.shift_right_logical(dv, 3)
                lane = lax.bitwise_and(dv, 7)
                plsc.store_scatter(pay, [iota16 + kk * 16, lane], ones16)

            pltpu.sync_copy(rows, acc.at[didx], add=True)
            pltpu.sync_copy(pay, cacc.at[cidx], add=True)

            @pl.loop(0, B // 16)
            def _(kk):
                dv = didx[pl.ds(kk * 16, 16)]
                lane = lax.bitwise_and(dv, 7)
                plsc.store_scatter(pay, [iota16 + kk * 16, lane], zeros16)

        plsc.subcore_barrier()

        for t in range(ROWS_PER_TILE // RCHUNK):
            r = row0 + t * RCHUNK
            pltpu.sync_copy(acc.at[pl.ds(r, RCHUNK)], zrows)
            pltpu.sync_copy(zrows, sum_hbm.at[c, pl.ds(r, RCHUNK)])
        pltpu.sync_copy(cacc.at[pl.ds(s * CPT, CPT)], rows)
        pltpu.sync_copy(rows, cnt_hbm.at[c, pl.ds(s * CPT, CPT)])

    return k(x, src, dst)


def _finish(x, p0, p1, c0, c1, W_l, b_l, W_r):
    R = 1000
    dn = (((1,), (1,)), ((), ()))

    def body(p0_ref, p1_ref, c0_ref, c1_ref, x_ref, wl_ref, wr_ref, b_ref,
             o_ref):
        cnt = jnp.maximum(c0_ref[...] + c1_ref[...], 1.0)
        agg = (p0_ref[...] + p1_ref[...]) / cnt
        acc = lax.dot_general(agg, wl_ref[...], dn,
                              precision=lax.Precision.HIGHEST,
                              preferred_element_type=jnp.float32)
        acc = acc + lax.dot_general(x_ref[...], wr_ref[...], dn,
                                    precision=lax.Precision.HIGHEST,
                                    preferred_element_type=jnp.float32)
        o_ref[...] = jnp.maximum(acc + b_ref[...], 0.0)

    return pl.pallas_call(
        body,
        grid=(N // R,),
        in_specs=[
            pl.BlockSpec((R, D), lambda i: (i, 0)),
            pl.BlockSpec((R, D), lambda i: (i, 0)),
            pl.BlockSpec((R, 1), lambda i: (i, 0)),
            pl.BlockSpec((R, 1), lambda i: (i, 0)),
            pl.BlockSpec((R, D), lambda i: (i, 0)),
            pl.BlockSpec((D, D), lambda i: (0, 0)),
            pl.BlockSpec((D, D), lambda i: (0, 0)),
            pl.BlockSpec((1, D), lambda i: (0, 0)),
        ],
        out_specs=pl.BlockSpec((R, D), lambda i: (i, 0)),
        out_shape=jax.ShapeDtypeStruct((N, D), jnp.float32),
    )(p0, p1, c0, c1, x, W_l, W_r, b_l.reshape(1, D))


def kernel(x, adj, W_l, b_l, W_r):
    src = adj[0].astype(jnp.int32)
    dst = adj[1].astype(jnp.int32)
    sums, cnts = _sc_segment_sum(x, src, dst)
    cc = cnts[:, :, :8].reshape(NC, NPAD)[:, :N]
    return _finish(x, sums[0, :N], sums[1, :N],
                   cc[0][:, None], cc[1][:, None], W_l, b_l, W_r)


# speedup vs baseline: 10.1080x; 1.9218x over previous
import dataclasses
import functools

import jax
import jax.numpy as jnp
from jax import lax
from jax.experimental import pallas as pl
from jax.experimental.pallas import tpu as pltpu
from jax.experimental.pallas import tpu_sc as plsc

N = 10000
NPAD = 10240
E = 320000
D = 128
NC = 2
NS = 16
NW = NC * NS
EPT = E // NW
B = 80
NB = EPT // B
PIPE = 3
NMAIN = (NB // PIPE) * PIPE
ROWS_PER_TILE = NPAD // NS
RCHUNK = 128
CR = NPAD // 16
CPT = CR // NS


def _sc_compiler_params():
    cp = pltpu.CompilerParams()
    fields = pltpu.CompilerParams.__dataclass_fields__
    if "needs_layout_passes" in fields:
        cp = dataclasses.replace(cp, needs_layout_passes=False)
    if "use_tc_tiling_on_sc" in fields:
        cp = dataclasses.replace(cp, use_tc_tiling_on_sc=False)
    return cp


def _sc_segment_sum(x, src, dst):
    mesh = plsc.VectorSubcoreMesh(core_axis_name="c", subcore_axis_name="s")

    @functools.partial(
        pl.kernel,
        compiler_params=_sc_compiler_params(),
        out_type=[
            jax.ShapeDtypeStruct((NC, NPAD, D), jnp.float32),
            jax.ShapeDtypeStruct((NC, CR, 16), jnp.float32),
        ],
        mesh=mesh,
        scratch_types=(
            [pltpu.VMEM((B,), jnp.int32)] * PIPE
            + [pltpu.VMEM((B,), jnp.int32)] * PIPE
            + [pltpu.VMEM((B, D), jnp.float32)] * PIPE
            + [
                pltpu.VMEM((CR, 16), jnp.float32),
                pltpu.VMEM((CR,), jnp.int32),
                pltpu.VMEM((CPT, 16), jnp.float32),
                pltpu.VMEM_SHARED((NPAD, D), jnp.float32),
                pltpu.VMEM_SHARED((CR, 16), jnp.float32),
            ]
            + [pltpu.SemaphoreType.DMA] * (4 * PIPE)
        ),
    )
    def k(x_hbm, src_hbm, dst_hbm, sum_hbm, cnt_hbm,
          x0, x1, x2, d0, d1, d2, r0, r1, r2,
          cnt_local, idxid, cbounce, acc, cacc, *sems):
        sidx = [x0, x1, x2]
        didx = [d0, d1, d2]
        rows = [r0, r1, r2]
        sem_x = sems[0:PIPE]
        sem_d = sems[PIPE:2 * PIPE]
        sem_g = sems[2 * PIPE:3 * PIPE]
        sem_s = sems[3 * PIPE:4 * PIPE]
        c = lax.axis_index("c")
        s = lax.axis_index("s")
        wid = c * NS + s
        base = wid * EPT
        ones16 = jnp.ones((16,), jnp.float32)
        zeros16 = jnp.zeros((16,), jnp.float32)
        iota16 = lax.iota(jnp.int32, 16)

        @pl.loop(0, CR // 16)
        def _(i):
            idxid[pl.ds(i * 16, 16)] = iota16 + i * 16

        @pl.loop(0, CR)
        def _(i):
            cnt_local[i, :] = zeros16

        @pl.loop(0, CPT)
        def _(i):
            cbounce[i, :] = zeros16

        @pl.loop(0, B)
        def _(i):
            @pl.loop(0, D // 16)
            def _(j):
                rows[0][i, pl.ds(j * 16, 16)] = zeros16

        row0 = s * ROWS_PER_TILE
        for t in range(ROWS_PER_TILE // B):
            pltpu.sync_copy(rows[0], acc.at[pl.ds(row0 + t * B, B)])
        pltpu.sync_copy(cbounce, cacc.at[pl.ds(s * CPT, CPT)])
        plsc.subcore_barrier()

        def load_idx(j, p):
            off = base + j * B
            pltpu.async_copy(src_hbm.at[pl.ds(off, B)], sidx[p], sem_x[p])
            pltpu.async_copy(dst_hbm.at[pl.ds(off, B)], didx[p], sem_d[p])

        def step(j, b, prefetch):
            p1 = (b + 1) % PIPE
            p2 = (b + 2) % PIPE

            pltpu.make_async_copy(
                x_hbm.at[sidx[b]], rows[b], sem_g[b]).wait()
            pltpu.make_async_copy(
                dst_hbm.at[pl.ds(base + j * B, B)], didx[b],
                sem_d[b]).wait()

            for kk in range(B // 16):
                dv = didx[b][pl.ds(kk * 16, 16)]
                crow = lax.shift_right_logical(dv, 4)
                ccol = lax.bitwise_and(dv, 15)
                plsc.addupdate_scatter(cnt_local, [crow, ccol], ones16)

            @pl.when(j >= 1)
            def _():
                pltpu.make_async_copy(
                    rows[p2], acc.at[didx[p2]], sem_s[p2]).wait()

            if prefetch:
                @pl.when(j + 2 < NB)
                def _():
                    load_idx(j + 2, p2)

                @pl.when(j + 1 < NB)
                def _():
                    pltpu.make_async_copy(
                        src_hbm.at[pl.ds(base + (j + 1) * B, B)], sidx[p1],
                        sem_x[p1]).wait()
                    pltpu.async_copy(x_hbm.at[sidx[p1]], rows[p1],
                                     sem_g[p1])

            pltpu.async_copy(rows[b], acc.at[didx[b]], sem_s[b], add=True)

        load_idx(0, 0)
        load_idx(1, 1)
        pltpu.make_async_copy(src_hbm.at[pl.ds(base, B)], sidx[0],
                              sem_x[0]).wait()
        pltpu.async_copy(x_hbm.at[sidx[0]], rows[0], sem_g[0])

        @pl.loop(0, NMAIN // PIPE)
        def _(jo):
            j0 = jo * PIPE
            for b in range(PIPE):
                step(j0 + b, b, True)

        for jt in range(NMAIN, NB):
            step(jt, jt % PIPE, True)

        bl = (NB - 1) % PIPE
        pltpu.make_async_copy(rows[bl], acc.at[didx[bl]], sem_s[bl]).wait()

        pltpu.sync_copy(cnt_local, cacc.at[idxid], add=True)
        plsc.subcore_barrier()

        for t in range(ROWS_PER_TILE // B):
            r = row0 + t * B
            pltpu.sync_copy(acc.at[pl.ds(r, B)], rows[0])
            pltpu.sync_copy(rows[0], sum_hbm.at[c, pl.ds(r, B)])
        pltpu.sync_copy(cacc.at[pl.ds(s * CPT, CPT)], cbounce)
        pltpu.sync_copy(cbounce, cnt_hbm.at[c, pl.ds(s * CPT, CPT)])

    return k(x, src, dst)


def _finish(x, p0, p1, c0, c1, W_l, b_l, W_r):
    R = 1000
    dn = (((1,), (1,)), ((), ()))

    def body(p0_ref, p1_ref, c0_ref, c1_ref, x_ref, wl_ref, wr_ref, b_ref,
             o_ref):
        cnt = jnp.maximum(c0_ref[...] + c1_ref[...], 1.0)
        agg = (p0_ref[...] + p1_ref[...]) / cnt
        acc = lax.dot_general(agg, wl_ref[...], dn,
                              precision=lax.Precision.HIGHEST,
                              preferred_element_type=jnp.float32)
        acc = acc + lax.dot_general(x_ref[...], wr_ref[...], dn,
                                    precision=lax.Precision.HIGHEST,
                                    preferred_element_type=jnp.float32)
        o_ref[...] = jnp.maximum(acc + b_ref[...], 0.0)

    return pl.pallas_call(
        body,
        grid=(N // R,),
        in_specs=[
            pl.BlockSpec((R, D), lambda i: (i, 0)),
            pl.BlockSpec((R, D), lambda i: (i, 0)),
            pl.BlockSpec((R, 1), lambda i: (i, 0)),
            pl.BlockSpec((R, 1), lambda i: (i, 0)),
            pl.BlockSpec((R, D), lambda i: (i, 0)),
            pl.BlockSpec((D, D), lambda i: (0, 0)),
            pl.BlockSpec((D, D), lambda i: (0, 0)),
            pl.BlockSpec((1, D), lambda i: (0, 0)),
        ],
        out_specs=pl.BlockSpec((R, D), lambda i: (i, 0)),
        out_shape=jax.ShapeDtypeStruct((N, D), jnp.float32),
    )(p0, p1, c0, c1, x, W_l, W_r, b_l.reshape(1, D))


def kernel(x, adj, W_l, b_l, W_r):
    src = adj[0].astype(jnp.int32)
    dst = adj[1].astype(jnp.int32)
    sums, cnts = _sc_segment_sum(x, src, dst)
    cc = cnts.reshape(NC, NPAD)[:, :N]
    return _finish(x, sums[0, :N], sums[1, :N],
                   cc[0][:, None], cc[1][:, None], W_l, b_l, W_r)
